# baseline (speedup 1.0000x reference)
import functools

import jax
import jax.numpy as jnp
from jax import lax
from jax.experimental import pallas as pl
from jax.experimental.pallas import tpu as pltpu
from jax.experimental.pallas import tpu_sc as plsc

_NC = 2
_NS = 16
_NW = _NC * _NS
_LANES = 16
_IDX_CHUNK = 128

_BATCH = 16384
_FACTORS = 32
_B_PER_W = _BATCH // _NW
_N_CHUNKS = _B_PER_W // _IDX_CHUNK


def _gmf_body(uids, iids, utab, itab, wv_hbm, bv_hbm, out,
              uidx, iidx, urows, irows, wv, bv, outv, sem):
    wid = lax.axis_index("s") * _NC + lax.axis_index("c")

    pltpu.sync_copy(uids.at[pl.ds(wid * _N_CHUNKS, _N_CHUNKS)], uidx)
    pltpu.sync_copy(iids.at[pl.ds(wid * _N_CHUNKS, _N_CHUNKS)], iidx)
    pltpu.sync_copy(wv_hbm, wv)
    pltpu.sync_copy(bv_hbm, bv)

    copies = []
    for c in range(_N_CHUNKS):
        copies.append(pltpu.async_copy(
            utab.at[uidx.at[c]], urows.at[pl.ds(c * _IDX_CHUNK, _IDX_CHUNK)], sem))
        copies.append(pltpu.async_copy(
            itab.at[iidx.at[c]], irows.at[pl.ds(c * _IDX_CHUNK, _IDX_CHUNK)], sem))
    for cp in copies:
        cp.wait()

    w_lo = wv[pl.ds(0, _LANES)]
    w_hi = wv[pl.ds(_LANES, _LANES)]
    bias = bv[...]
    lane = lax.iota(jnp.int32, _LANES)

    def group(g, carry):
        base = g * _LANES
        acc = jnp.zeros((_LANES,), jnp.float32)
        for r in range(_LANES):
            row = base + r
            u_lo = urows[row, pl.ds(0, _LANES)]
            u_hi = urows[row, pl.ds(_LANES, _LANES)]
            i_lo = irows[row, pl.ds(0, _LANES)]
            i_hi = irows[row, pl.ds(_LANES, _LANES)]
            p = u_lo * i_lo * w_lo + u_hi * i_hi * w_hi
            acc = jnp.where(lane == r, jnp.sum(p), acc)
        logits = acc + bias
        outv[pl.ds(base, _LANES)] = 1.0 / (1.0 + jnp.exp(-logits))
        return carry

    lax.fori_loop(0, _B_PER_W // _LANES, group, 0, unroll=False)

    pltpu.sync_copy(outv, out.at[pl.ds(wid * _B_PER_W, _B_PER_W)])


_gmf_call = functools.partial(
    pl.kernel,
    out_type=jax.ShapeDtypeStruct((_BATCH,), jnp.float32),
    mesh=plsc.VectorSubcoreMesh(core_axis_name="c", subcore_axis_name="s"),
    compiler_params=pltpu.CompilerParams(
        needs_layout_passes=False, use_tc_tiling_on_sc=False),
    scratch_types=[
        pltpu.VMEM((_N_CHUNKS, _IDX_CHUNK), jnp.int32),
        pltpu.VMEM((_N_CHUNKS, _IDX_CHUNK), jnp.int32),
        pltpu.VMEM((_B_PER_W, _FACTORS), jnp.float32),
        pltpu.VMEM((_B_PER_W, _FACTORS), jnp.float32),
        pltpu.VMEM((_FACTORS,), jnp.float32),
        pltpu.VMEM((_LANES,), jnp.float32),
        pltpu.VMEM((_B_PER_W,), jnp.float32),
        pltpu.SemaphoreType.DMA,
    ],
)(_gmf_body)


@jax.jit
def kernel(user_ids, item_ids, user_table, item_table, W, b):
    uids = user_ids.astype(jnp.int32).reshape(_NW * _N_CHUNKS, _IDX_CHUNK)
    iids = item_ids.astype(jnp.int32).reshape(_NW * _N_CHUNKS, _IDX_CHUNK)
    wv = W.reshape(_FACTORS).astype(jnp.float32)
    bv = jnp.broadcast_to(b.reshape(()), (_LANES,)).astype(jnp.float32)
    out = _gmf_call(uids, iids, user_table, item_table, wv, bv)
    return out.reshape(_BATCH, 1)
